# baseline (speedup 1.0000x reference)
import functools

import jax
import jax.numpy as jnp
from jax import lax
from jax.experimental import pallas as pl
from jax.experimental.pallas import tpu as pltpu
from jax.experimental.pallas import tpu_sc as plsc

_NC = 2
_NS = 16
_NW = _NC * _NS
_LANES = 128
_VL = 16
_NB = 4
_SLAB = 8
_SUP = 4


@functools.cache
def _make_lookup(batch: int, hist: int, vocab: int, dim: int):
  pack = _LANES // dim
  n_blocks = batch // (_NW * _NB)
  n_bodies = n_blocks // 8
  assert n_blocks * _NW * _NB == batch and n_bodies * 8 == n_blocks
  assert n_bodies >= 3
  srows = _SUP * _NB * hist // pack
  brows = _NB * hist // pack
  out_rows = batch * hist // pack
  tile_rows = n_blocks * brows

  mesh = plsc.VectorSubcoreMesh(core_axis_name="c", subcore_axis_name="s")

  @functools.partial(
      pl.kernel,
      mesh=mesh,
      out_type=jax.ShapeDtypeStruct((out_rows, _LANES), jnp.float32),
      scratch_types=[
          pltpu.VMEM((2, _SLAB, hist), jnp.int32),
          pltpu.VMEM((_NB, hist, _LANES), jnp.float32),
          pltpu.VMEM((_NB, hist, _LANES), jnp.float32),
          pltpu.VMEM((srows, _LANES), jnp.float32),
          pltpu.VMEM((srows, _LANES), jnp.float32),
          pltpu.SemaphoreType.DMA,
          pltpu.SemaphoreType.DMA,
          pltpu.SemaphoreType.DMA,
          pltpu.SemaphoreType.DMA,
      ],
  )
  def body(
      idx_hbm,
      table_hbm,
      out_hbm,
      idx_v,
      rows0,
      rows1,
      st0,
      st1,
      sem_i,
      sem_g,
      sem_o0,
      sem_o1,
  ):
    wid = lax.axis_index("s") * _NC + lax.axis_index("c")
    idx_base = wid * n_blocks * _NB
    row_base = wid * tile_rows

    def fire_idx(si, sb):
      off = pl.multiple_of(idx_base + si * _SLAB, _SLAB)
      pltpu.async_copy(idx_hbm.at[pl.ds(off, _SLAB)], idx_v.at[sb], sem_i)

    def drain_idx(sb):
      pltpu.make_async_copy(
          idx_hbm.at[pl.ds(0, _SLAB)], idx_v.at[sb], sem_i
      ).wait()

    def fire_g(rows, sb, half):
      for r in range(_NB):
        pltpu.async_copy(
            table_hbm.at[idx_v.at[sb].at[half * _NB + r]],
            rows.at[r],
            sem_g,
        )

    def drain_g(rows, sb, half):
      for r in range(_NB):
        pltpu.make_async_copy(
            table_hbm.at[idx_v.at[sb].at[half * _NB + r]],
            rows.at[r],
            sem_g,
        ).wait()

    def repack(rows, st, slot):
      tail = hist % 4

      def pack_rh(r, e, t):
        row_static = slot * brows + 12 * r + (2 * r + e) // 4
        lane = dim * ((2 * r + e) % 4)
        for v in range(dim // _VL):
          st[row_static + t, pl.ds(lane + v * _VL, _VL)] = rows[
              r, 4 * t + e, pl.ds(v * _VL, _VL)
          ]

      def per_t(t, _):
        for r in range(_NB):
          for e in range(4):
            pack_rh(r, e, t)
        return 0

      lax.fori_loop(0, hist // 4, per_t, 0, unroll=False)
      for r in range(_NB):
        for e in range(tail):
          pack_rh(r, e, hist // 4)

    def fire_w(u, st, sem):
      off = pl.multiple_of(row_base + u * srows, _SLAB)
      pltpu.async_copy(st, out_hbm.at[pl.ds(off, srows)], sem)

    def drain_w(st, sem):
      pltpu.make_async_copy(st, out_hbm.at[pl.ds(0, srows)], sem).wait()

    fire_idx(0, 0)
    drain_idx(0)
    fire_g(rows0, 0, 0)
    fire_idx(1, 1)
    drain_g(rows0, 0, 0)
    fire_g(rows1, 0, 1)
    repack(rows0, st0, 0)
    drain_idx(1)
    drain_g(rows1, 0, 1)
    fire_g(rows0, 1, 0)
    fire_idx(2, 0)
    repack(rows1, st0, 1)
    drain_g(rows0, 1, 0)
    fire_g(rows1, 1, 1)
    repack(rows0, st0, 2)
    drain_idx(0)
    drain_g(rows1, 1, 1)
    fire_g(rows0, 0, 0)
    fire_idx(3, 1)
    repack(rows1, st0, 3)
    fire_w(0, st0, sem_o0)
    drain_g(rows0, 0, 0)
    fire_g(rows1, 0, 1)
    repack(rows0, st1, 0)
    drain_idx(1)
    drain_g(rows1, 0, 1)
    fire_g(rows0, 1, 0)
    fire_idx(4, 0)
    repack(rows1, st1, 1)
    drain_g(rows0, 1, 0)
    fire_g(rows1, 1, 1)
    repack(rows0, st1, 2)

    def step(sp, _):
      drain_idx(0)
      drain_g(rows1, 1, 1)
      fire_g(rows0, 0, 0)
      fire_idx(4 * sp + 1, 1)
      repack(rows1, st1, 3)
      fire_w(2 * sp - 1, st1, sem_o1)
      drain_g(rows0, 0, 0)
      fire_g(rows1, 0, 1)
      drain_w(st0, sem_o0)
      repack(rows0, st0, 0)
      drain_idx(1)
      drain_g(rows1, 0, 1)
      fire_g(rows0, 1, 0)
      fire_idx(4 * sp + 2, 0)
      repack(rows1, st0, 1)
      drain_g(rows0, 1, 0)
      fire_g(rows1, 1, 1)
      repack(rows0, st0, 2)
      drain_idx(0)
      drain_g(rows1, 1, 1)
      fire_g(rows0, 0, 0)
      fire_idx(4 * sp + 3, 1)
      repack(rows1, st0, 3)
      fire_w(2 * sp, st0, sem_o0)
      drain_g(rows0, 0, 0)
      fire_g(rows1, 0, 1)
      drain_w(st1, sem_o1)
      repack(rows0, st1, 0)
      drain_idx(1)
      drain_g(rows1, 0, 1)
      fire_g(rows0, 1, 0)
      fire_idx(4 * sp + 4, 0)
      repack(rows1, st1, 1)
      drain_g(rows0, 1, 0)
      fire_g(rows1, 1, 1)
      repack(rows0, st1, 2)
      return 0

    lax.fori_loop(1, n_bodies - 1, step, 0, unroll=False)

    sp = n_bodies - 1
    drain_idx(0)
    drain_g(rows1, 1, 1)
    fire_g(rows0, 0, 0)
    fire_idx(4 * sp + 1, 1)
    repack(rows1, st1, 3)
    fire_w(2 * sp - 1, st1, sem_o1)
    drain_g(rows0, 0, 0)
    fire_g(rows1, 0, 1)
    drain_w(st0, sem_o0)
    repack(rows0, st0, 0)
    drain_idx(1)
    drain_g(rows1, 0, 1)
    fire_g(rows0, 1, 0)
    fire_idx(4 * sp + 2, 0)
    repack(rows1, st0, 1)
    drain_g(rows0, 1, 0)
    fire_g(rows1, 1, 1)
    repack(rows0, st0, 2)
    drain_idx(0)
    drain_g(rows1, 1, 1)
    fire_g(rows0, 0, 0)
    fire_idx(4 * sp + 3, 1)
    repack(rows1, st0, 3)
    fire_w(2 * sp, st0, sem_o0)
    drain_g(rows0, 0, 0)
    fire_g(rows1, 0, 1)
    drain_w(st1, sem_o1)
    repack(rows0, st1, 0)
    drain_idx(1)
    drain_g(rows1, 0, 1)
    fire_g(rows0, 1, 0)
    repack(rows1, st1, 1)
    drain_g(rows0, 1, 0)
    fire_g(rows1, 1, 1)
    repack(rows0, st1, 2)
    drain_g(rows1, 1, 1)
    repack(rows1, st1, 3)
    fire_w(2 * sp + 1, st1, sem_o1)
    drain_w(st0, sem_o0)
    drain_w(st1, sem_o1)

  return body


def kernel(input, lookup):
  batch, hist = input.shape
  vocab, dim = lookup.shape
  table = jnp.pad(lookup, ((0, 0), (0, _LANES - dim)))
  out2d = _make_lookup(batch, hist, vocab, dim)(input, table)
  return out2d.reshape(batch, hist, dim)


# speedup vs baseline: 1.0950x; 1.0950x over previous
import functools

import jax
import jax.numpy as jnp
from jax import lax
from jax.experimental import pallas as pl
from jax.experimental.pallas import tpu as pltpu
from jax.experimental.pallas import tpu_sc as plsc

_NC = 2
_NS = 16
_NW = _NC * _NS
_LANES = 128
_VL = 16
_NB = 4
_SLAB = 8


@functools.cache
def _make_lookup(batch: int, hist: int, vocab: int, dim: int):
  n_blocks = batch // (_NW * _NB)
  n_slabs = n_blocks // 2
  assert n_blocks * _NW * _NB == batch and n_slabs * 2 == n_blocks
  assert n_slabs >= 3

  mesh = plsc.VectorSubcoreMesh(core_axis_name="c", subcore_axis_name="s")

  @functools.partial(
      pl.kernel,
      mesh=mesh,
      out_type=jax.ShapeDtypeStruct((batch, hist, dim), jnp.float32),
      scratch_types=[
          pltpu.VMEM((2, _SLAB, hist), jnp.int32),
          pltpu.VMEM((_NB, hist, _LANES), jnp.float32),
          pltpu.VMEM((_NB, hist, _LANES), jnp.float32),
          pltpu.VMEM((_NB, hist, dim), jnp.float32),
          pltpu.VMEM((_NB, hist, dim), jnp.float32),
          pltpu.SemaphoreType.DMA,
          pltpu.SemaphoreType.DMA,
          pltpu.SemaphoreType.DMA,
          pltpu.SemaphoreType.DMA,
      ],
  )
  def body(
      idx_hbm,
      table_hbm,
      out_hbm,
      idx_v,
      rows0,
      rows1,
      comp0,
      comp1,
      sem_i,
      sem_g,
      sem_o0,
      sem_o1,
  ):
    wid = lax.axis_index("s") * _NC + lax.axis_index("c")
    base = wid * n_blocks * _NB

    def fire_idx(s, sb):
      off = pl.multiple_of(base + s * _SLAB, _SLAB)
      pltpu.async_copy(idx_hbm.at[pl.ds(off, _SLAB)], idx_v.at[sb], sem_i)

    def drain_idx(sb):
      pltpu.make_async_copy(
          idx_hbm.at[pl.ds(0, _SLAB)], idx_v.at[sb], sem_i
      ).wait()

    def fire_gathers(rows, sb, half):
      for r in range(_NB):
        pltpu.async_copy(
            table_hbm.at[idx_v.at[sb].at[half * _NB + r]],
            rows.at[r],
            sem_g,
        )

    def drain_gathers(rows, sb, half):
      for r in range(_NB):
        pltpu.make_async_copy(
            table_hbm.at[idx_v.at[sb].at[half * _NB + r]],
            rows.at[r],
            sem_g,
        ).wait()

    def repack(rows, comp):
      def per_r(r, _):
        for h in range(hist):
          for v in range(dim // _VL):
            comp[r, h, pl.ds(v * _VL, _VL)] = rows[r, h, pl.ds(v * _VL, _VL)]
        return 0

      lax.fori_loop(0, _NB, per_r, 0, unroll=False)

    def fire_write(j, comp, sem):
      off = base + j * _NB
      pltpu.async_copy(comp, out_hbm.at[pl.ds(off, _NB)], sem)

    def drain_write(comp, sem):
      pltpu.make_async_copy(comp, out_hbm.at[pl.ds(0, _NB)], sem).wait()

    fire_idx(0, 0)
    drain_idx(0)
    fire_gathers(rows0, 0, 0)
    fire_idx(1, 1)
    drain_gathers(rows0, 0, 0)
    fire_gathers(rows1, 0, 1)
    repack(rows0, comp0)
    fire_write(0, comp0, sem_o0)
    drain_idx(1)
    drain_gathers(rows1, 0, 1)
    fire_gathers(rows0, 1, 0)
    repack(rows1, comp1)
    fire_write(1, comp1, sem_o1)
    drain_gathers(rows0, 1, 0)
    fire_gathers(rows1, 1, 1)
    fire_idx(2, 0)
    drain_write(comp0, sem_o0)
    repack(rows0, comp0)
    fire_write(2, comp0, sem_o0)

    def step(s, _):
      sb = s % 2
      drain_idx(sb)
      drain_gathers(rows1, sb, 1)
      fire_gathers(rows0, sb, 0)
      drain_write(comp1, sem_o1)
      repack(rows1, comp1)
      fire_write(2 * s - 1, comp1, sem_o1)
      drain_gathers(rows0, sb, 0)
      fire_gathers(rows1, sb, 1)
      fire_idx(s + 1, 1 - sb)
      drain_write(comp0, sem_o0)
      repack(rows0, comp0)
      fire_write(2 * s, comp0, sem_o0)
      return 0

    lax.fori_loop(2, n_slabs - 1, step, 0, unroll=False)

    s = n_slabs - 1
    sb = s % 2
    drain_idx(sb)
    drain_gathers(rows1, sb, 1)
    fire_gathers(rows0, sb, 0)
    drain_write(comp1, sem_o1)
    repack(rows1, comp1)
    fire_write(2 * s - 1, comp1, sem_o1)
    drain_gathers(rows0, sb, 0)
    fire_gathers(rows1, sb, 1)
    drain_write(comp0, sem_o0)
    repack(rows0, comp0)
    fire_write(2 * s, comp0, sem_o0)
    drain_gathers(rows1, sb, 1)
    drain_write(comp1, sem_o1)
    repack(rows1, comp1)
    fire_write(2 * s + 1, comp1, sem_o1)
    drain_write(comp0, sem_o0)
    drain_write(comp1, sem_o1)

  return body


def kernel(input, lookup):
  batch, hist = input.shape
  vocab, dim = lookup.shape
  table = jnp.pad(lookup, ((0, 0), (0, _LANES - dim)))
  return _make_lookup(batch, hist, vocab, dim)(input, table)
